# baseline (speedup 1.0000x reference)
import functools

import jax
import jax.numpy as jnp
from jax.experimental import pallas as pl
from jax.experimental.pallas import tpu as pltpu

N_ATOMS = 4096
N_FEAT = 128
N_BOND = 4
N_OUT = 32
BM = 128


def _molconv_kernel(af_ref, wt_ref, b_ref, bond_a_ref, bond_b_ref,
                    out_ref, h_ref):
    @pl.when(pl.program_id(0) == 0)
    def _compute_h():
        h = jnp.dot(af_ref[...], wt_ref[...], preferred_element_type=jnp.float32)
        h = (h + b_ref[...]).astype(jnp.bfloat16)
        lane = jax.lax.broadcasted_iota(jnp.int32, (N_ATOMS, N_FEAT), 1)
        for bt in range(N_BOND):
            mask = (lane >= bt * N_OUT) & (lane < (bt + 1) * N_OUT)
            h_ref[pl.ds(bt * N_ATOMS, N_ATOMS), :] = jnp.where(
                mask, h, jnp.bfloat16(0))

    hh = h_ref[...]
    for off, bond in ((0, bond_a_ref[...]), (BM, bond_b_ref[...])):
        y = jnp.dot(bond.astype(jnp.bfloat16), hh,
                    preferred_element_type=jnp.float32)
        out_ref[pl.ds(off, BM), :] = (
            y[:, :N_OUT] + y[:, N_OUT:2 * N_OUT]
            + y[:, 2 * N_OUT:3 * N_OUT] + y[:, 3 * N_OUT:])


@functools.partial(jax.jit, static_argnames=())
def kernel(atom_features, bond_info, W, b):
    n = atom_features.shape[0]
    wt = W.T
    b2 = b.reshape(1, N_BOND * N_OUT)
    grid = (n // (2 * BM),)
    return pl.pallas_call(
        _molconv_kernel,
        grid=grid,
        in_specs=[
            pl.BlockSpec((n, N_FEAT), lambda i: (0, 0)),
            pl.BlockSpec((N_FEAT, N_BOND * N_OUT), lambda i: (0, 0)),
            pl.BlockSpec((1, N_BOND * N_OUT), lambda i: (0, 0)),
            pl.BlockSpec((BM, N_BOND * n), lambda i: (2 * i, 0)),
            pl.BlockSpec((BM, N_BOND * n), lambda i: (2 * i + 1, 0)),
        ],
        out_specs=pl.BlockSpec((2 * BM, N_OUT), lambda i: (i, 0)),
        out_shape=jax.ShapeDtypeStruct((n, N_OUT), jnp.float32),
        scratch_shapes=[pltpu.VMEM((N_BOND * n, N_FEAT), jnp.bfloat16)],
    )(atom_features, wt, b2, bond_info, bond_info)


# speedup vs baseline: 1.0265x; 1.0265x over previous
import functools

import jax
import jax.numpy as jnp
from jax.experimental import pallas as pl
from jax.experimental.pallas import tpu as pltpu

N_ATOMS = 4096
N_FEAT = 128
N_BOND = 4
N_OUT = 32
BM = 256


def _molconv_kernel(af_ref, wt_ref, b_ref, bond_ref, out_ref, h_ref):
    @pl.when(pl.program_id(0) == 0)
    def _compute_h():
        h = jnp.dot(af_ref[...], wt_ref[...], preferred_element_type=jnp.float32)
        h = (h + b_ref[...]).astype(jnp.bfloat16)
        lane = jax.lax.broadcasted_iota(jnp.int32, (N_ATOMS, N_FEAT), 1)
        for bt in range(N_BOND):
            mask = (lane >= bt * N_OUT) & (lane < (bt + 1) * N_OUT)
            h_ref[pl.ds(bt * N_ATOMS, N_ATOMS), :] = jnp.where(
                mask, h, jnp.bfloat16(0))

    y = jax.lax.dot_general(
        h_ref[...],
        bond_ref[...].astype(jnp.bfloat16),
        dimension_numbers=(((0,), (1,)), ((), ())),
        preferred_element_type=jnp.float32,
    )
    yt = y.T
    out_ref[...] = (yt[:, :N_OUT] + yt[:, N_OUT:2 * N_OUT]
                    + yt[:, 2 * N_OUT:3 * N_OUT] + yt[:, 3 * N_OUT:])


@functools.partial(jax.jit, static_argnames=())
def kernel(atom_features, bond_info, W, b):
    n = atom_features.shape[0]
    wt = W.T
    b2 = b.reshape(1, N_BOND * N_OUT)
    grid = (n // BM,)
    return pl.pallas_call(
        _molconv_kernel,
        grid=grid,
        in_specs=[
            pl.BlockSpec((n, N_FEAT), lambda i: (0, 0)),
            pl.BlockSpec((N_FEAT, N_BOND * N_OUT), lambda i: (0, 0)),
            pl.BlockSpec((1, N_BOND * N_OUT), lambda i: (0, 0)),
            pl.BlockSpec((BM, N_BOND * n), lambda i: (i, 0)),
        ],
        out_specs=pl.BlockSpec((BM, N_OUT), lambda i: (i, 0)),
        out_shape=jax.ShapeDtypeStruct((n, N_OUT), jnp.float32),
        scratch_shapes=[pltpu.VMEM((N_BOND * n, N_FEAT), jnp.bfloat16)],
    )(atom_features, wt, b2, bond_info)


# speedup vs baseline: 1.0299x; 1.0033x over previous
import functools

import jax
import jax.numpy as jnp
from jax.experimental import pallas as pl
from jax.experimental.pallas import tpu as pltpu

N_ATOMS = 4096
N_FEAT = 128
N_BOND = 4
N_OUT = 32
BM = 256


def _molconv_kernel(af_ref, wt_ref, b_ref, bond_ref, out_ref, h_ref):
    @pl.when(pl.program_id(0) == 0)
    def _compute_h():
        h = jnp.dot(af_ref[...], wt_ref[...], preferred_element_type=jnp.float32)
        h = (h + b_ref[...]).astype(jnp.bfloat16)
        lane = jax.lax.broadcasted_iota(jnp.int32, (N_ATOMS, N_FEAT), 1)
        for bt in range(N_BOND):
            mask = (lane >= bt * N_OUT) & (lane < (bt + 1) * N_OUT)
            h_ref[pl.ds(bt * N_ATOMS, N_ATOMS), :] = jnp.where(
                mask, h, jnp.bfloat16(0))

    y = jax.lax.dot_general(
        h_ref[...],
        bond_ref[...].astype(jnp.bfloat16),
        dimension_numbers=(((0,), (1,)), ((), ())),
        preferred_element_type=jnp.float32,
    )
    z = (y[:N_OUT, :] + y[N_OUT:2 * N_OUT, :]
         + y[2 * N_OUT:3 * N_OUT, :] + y[3 * N_OUT:, :])
    out_ref[...] = z.T


@functools.partial(jax.jit, static_argnames=())
def kernel(atom_features, bond_info, W, b):
    n = atom_features.shape[0]
    wt = W.T
    b2 = b.reshape(1, N_BOND * N_OUT)
    grid = (n // BM,)
    return pl.pallas_call(
        _molconv_kernel,
        grid=grid,
        in_specs=[
            pl.BlockSpec((n, N_FEAT), lambda i: (0, 0)),
            pl.BlockSpec((N_FEAT, N_BOND * N_OUT), lambda i: (0, 0)),
            pl.BlockSpec((1, N_BOND * N_OUT), lambda i: (0, 0)),
            pl.BlockSpec((BM, N_BOND * n), lambda i: (i, 0)),
        ],
        out_specs=pl.BlockSpec((BM, N_OUT), lambda i: (i, 0)),
        out_shape=jax.ShapeDtypeStruct((n, N_OUT), jnp.float32),
        scratch_shapes=[pltpu.VMEM((N_BOND * n, N_FEAT), jnp.bfloat16)],
    )(atom_features, wt, b2, bond_info)
